# Initial kernel scaffold; baseline (speedup 1.0000x reference)
import jax
import jax.numpy as jnp
from jax.experimental import pallas as pl


def kernel(x, edge_index, W1, b1, WV1, W2, b2, WV2, Wc, bc):
    raise NotImplementedError("write your pallas kernel here")


# speedup vs baseline: 2.4970x; 2.4970x over previous
import functools
import jax
import jax.numpy as jnp
from jax.experimental import pallas as pl
from jax.experimental.pallas import tpu as pltpu

N = 10000
E = 320000
D_H = 128
D_OUT = 64
BETA = 0.1


def _l2n(h):
    return h / (jnp.linalg.norm(h, axis=1, keepdims=True) + 1e-12)


def _gcn_jnp(x, src, dst, dinv, W):
    xw = x @ W
    y = xw * dinv[:, None]
    msg = jnp.take(y, src, axis=0)
    agg = jnp.zeros((N, W.shape[1]), xw.dtype).at[dst].add(msg)
    return (agg + y) * dinv[:, None]


def _post_body(s_ref, hn_ref, m_ref, b_ref, o_ref):
    z = s_ref[...] + jnp.dot(hn_ref[...], m_ref[...],
                             preferred_element_type=jnp.float32) + b_ref[...]
    z = jnp.maximum(z, 0.0)
    nrm = jnp.sqrt(jnp.sum(z * z, axis=1, keepdims=True))
    o_ref[...] = z / (nrm + 1e-12)


def _layer_post(z_local_nb, h_norm, M, b):
    blk = 1000
    grid = N // blk
    return pl.pallas_call(
        _post_body,
        grid=(grid,),
        in_specs=[
            pl.BlockSpec((blk, D_H), lambda i: (i, 0)),
            pl.BlockSpec((blk, D_H), lambda i: (i, 0)),
            pl.BlockSpec((D_H, D_H), lambda i: (0, 0)),
            pl.BlockSpec((1, D_H), lambda i: (0, 0)),
        ],
        out_specs=pl.BlockSpec((blk, D_H), lambda i: (i, 0)),
        out_shape=jax.ShapeDtypeStruct((N, D_H), jnp.float32),
    )(z_local_nb, h_norm, M, b.reshape(1, D_H))


def kernel(x, edge_index, W1, b1, WV1, W2, b2, WV2, Wc, bc):
    src = edge_index[0].astype(jnp.int32)
    dst = edge_index[1].astype(jnp.int32)
    deg = jnp.zeros((N,), jnp.float32).at[dst].add(1.0) + 1.0
    dinv = jax.lax.rsqrt(deg)

    h = x
    for (W, b, WV) in ((W1, b1, WV1), (W2, b2, WV2)):
        z_local = _gcn_jnp(h, src, dst, dinv, W)
        hn = _l2n(h)
        HtH = hn.T @ hn
        M = (BETA / N) * (HtH @ WV)
        h = _layer_post(z_local, hn, M, b)

    out = h @ Wc + bc
    qel = jnp.array(0.0, dtype=jnp.float32)
    return (out, qel)


# speedup vs baseline: 27.3530x; 10.9542x over previous
import functools
import jax
import jax.numpy as jnp
from jax import lax
from jax.experimental import pallas as pl
from jax.experimental.pallas import tpu as pltpu
from jax.experimental.pallas import tpu_sc as plsc

N = 10000
E = 320000
D_H = 128
D_OUT = 64
BETA = 0.1

NC = 2
NS = 16
NW = NC * NS
EPT = E // NW
CHUNK = 80
NCH = EPT // CHUNK
NPAD = 10240
STRIPE = NPAD // NS

_sc_mesh = plsc.VectorSubcoreMesh(core_axis_name="c", subcore_axis_name="s")
_sc_params = pltpu.CompilerParams(needs_layout_passes=False)


def _deg_body(dst_hbm, out_hbm, dstv, hist):
    cid = lax.axis_index("c")
    sid = lax.axis_index("s")
    wid = sid * NC + cid

    z16 = jnp.zeros((16,), jnp.float32)

    def zero_row(i, _):
        hist[pl.ds(i * 16, 16)] = z16
        return 0

    lax.fori_loop(0, N // 16, zero_row, 0)

    pltpu.sync_copy(dst_hbm.at[pl.ds(wid * EPT, EPT)], dstv)

    ones = jnp.full((16,), 1.0, jnp.float32)

    def count(i, _):
        d = dstv[pl.ds(i * 16, 16)]
        plsc.addupdate_scatter(hist, [d], ones)
        return 0

    lax.fori_loop(0, EPT // 16, count, 0)
    pltpu.sync_copy(hist, out_hbm.at[wid])


def _deg_partials(dst):
    return pl.kernel(
        _deg_body,
        out_type=jax.ShapeDtypeStruct((NW, N), jnp.float32),
        mesh=_sc_mesh,
        scratch_types=[
            pltpu.VMEM((EPT,), jnp.int32),
            pltpu.VMEM((N,), jnp.float32),
        ],
        compiler_params=_sc_params,
    )(dst)


def _scatter_body(y_hbm, src_hbm, dst_hbm, zeros_hbm, out_hbm,
                  srcv, dstv, ssm0, ssm1, dsm0, dsm1, rows0, rows1,
                  acc, sem0, sem1):
    cid = lax.axis_index("c")
    sid = lax.axis_index("s")
    wid = sid * NC + cid
    base = wid * EPT

    pltpu.sync_copy(src_hbm.at[pl.ds(base, EPT)], srcv)
    pltpu.sync_copy(dst_hbm.at[pl.ds(base, EPT)], dstv)

    def fill(small, big, c):
        for t in range(CHUNK // 16):
            small[pl.ds(t * 16, 16)] = big[pl.ds(c * CHUNK + t * 16, 16)]

    def start(c, ssm, rows, sem):
        fill(ssm, srcv, c)
        pltpu.async_copy(y_hbm.at[ssm], rows, sem)

    def finish(ssm, rows, sem):
        pltpu.make_async_copy(y_hbm.at[ssm], rows, sem).wait()

    def scatter(c, dsm, rows):
        fill(dsm, dstv, c)
        pltpu.sync_copy(rows, acc.at[dsm], add=True)

    start(0, ssm0, rows0, sem0)

    pltpu.sync_copy(zeros_hbm.at[pl.ds(sid * STRIPE, STRIPE)],
                    acc.at[pl.ds(sid * STRIPE, STRIPE)])
    plsc.subcore_barrier()

    def body(i, _):
        c = 2 * i
        start(c + 1, ssm1, rows1, sem1)
        finish(ssm0, rows0, sem0)
        scatter(c, dsm0, rows0)
        start(c + 2, ssm0, rows0, sem0)
        finish(ssm1, rows1, sem1)
        scatter(c + 1, dsm1, rows1)
        return 0

    lax.fori_loop(0, (NCH - 1) // 2, body, 0)
    finish(ssm0, rows0, sem0)
    scatter(NCH - 1, dsm0, rows0)

    plsc.subcore_barrier()
    pltpu.sync_copy(acc.at[pl.ds(sid * STRIPE, STRIPE)],
                    out_hbm.at[cid, pl.ds(sid * STRIPE, STRIPE)])


def _edge_scatter(y, src, dst, zeros_nd):
    return pl.kernel(
        _scatter_body,
        out_type=jax.ShapeDtypeStruct((NC, NPAD, D_H), jnp.float32),
        mesh=_sc_mesh,
        scratch_types=[
            pltpu.VMEM((EPT,), jnp.int32),
            pltpu.VMEM((EPT,), jnp.int32),
            pltpu.VMEM((CHUNK,), jnp.int32),
            pltpu.VMEM((CHUNK,), jnp.int32),
            pltpu.VMEM((CHUNK,), jnp.int32),
            pltpu.VMEM((CHUNK,), jnp.int32),
            pltpu.VMEM((CHUNK, D_H), jnp.float32),
            pltpu.VMEM((CHUNK, D_H), jnp.float32),
            pltpu.VMEM_SHARED((NPAD, D_H), jnp.float32),
            pltpu.SemaphoreType.DMA,
            pltpu.SemaphoreType.DMA,
        ],
        compiler_params=_sc_params,
    )(y, src, dst, zeros_nd)


def _post_body(s_ref, hn_ref, m_ref, b_ref, mask_ref, o_ref):
    z = s_ref[...] + jnp.dot(hn_ref[...], m_ref[...],
                             preferred_element_type=jnp.float32) + b_ref[...]
    z = jnp.maximum(z, 0.0)
    nrm = jnp.sqrt(jnp.sum(z * z, axis=1, keepdims=True))
    o_ref[...] = (z / (nrm + 1e-12)) * mask_ref[...]


def _layer_post(z_local, h_norm, M, b, mask):
    blk = 1024
    return pl.pallas_call(
        _post_body,
        grid=(NPAD // blk,),
        in_specs=[
            pl.BlockSpec((blk, D_H), lambda i: (i, 0)),
            pl.BlockSpec((blk, D_H), lambda i: (i, 0)),
            pl.BlockSpec((D_H, D_H), lambda i: (0, 0)),
            pl.BlockSpec((1, D_H), lambda i: (0, 0)),
            pl.BlockSpec((blk, 1), lambda i: (i, 0)),
        ],
        out_specs=pl.BlockSpec((blk, D_H), lambda i: (i, 0)),
        out_shape=jax.ShapeDtypeStruct((NPAD, D_H), jnp.float32),
    )(z_local, h_norm, M, b.reshape(1, D_H), mask)


def _l2n(h):
    return h / (jnp.linalg.norm(h, axis=1, keepdims=True) + 1e-12)


def kernel(x, edge_index, W1, b1, WV1, W2, b2, WV2, Wc, bc):
    src = edge_index[0].astype(jnp.int32)
    dst = edge_index[1].astype(jnp.int32)

    degp = _deg_partials(dst)
    deg = degp.sum(axis=0) + 1.0
    dinv = lax.rsqrt(deg)
    dinv = jnp.concatenate([dinv, jnp.ones((NPAD - N,), jnp.float32)])
    zeros_nd = jnp.zeros((NPAD, D_H), jnp.float32)
    mask = (jnp.arange(NPAD) < N).astype(jnp.float32)[:, None]

    h = jnp.zeros((NPAD, D_H), jnp.float32).at[:N].set(x)
    for (W, b, WV) in ((W1, b1, WV1), (W2, b2, WV2)):
        y = (h @ W) * dinv[:, None]
        Sp = _edge_scatter(y, src, dst, zeros_nd)
        z_local = (Sp[0] + Sp[1] + y) * dinv[:, None]
        hn = _l2n(h)
        HtH = hn.T @ hn
        M = (BETA / N) * (HtH @ WV)
        h = _layer_post(z_local, hn, M, b, mask)

    out = h[:N] @ Wc + bc
    qel = jnp.array(0.0, dtype=jnp.float32)
    return (out, qel)
